# Initial kernel scaffold; baseline (speedup 1.0000x reference)
import jax
import jax.numpy as jnp
from jax.experimental import pallas as pl


def kernel(x, adj, W1, b1, W2, b2, W3, b3):
    raise NotImplementedError("write your pallas kernel here")


# speedup vs baseline: 1.2854x; 1.2854x over previous
import jax
import jax.numpy as jnp
from jax.experimental import pallas as pl
from jax.experimental.pallas import tpu as pltpu
from functools import partial

N = 4096
D_IN = 256
D_OUT = 256
HIDDEN = 128
ITRS = 5
TILE = 512
T = N // TILE


def _gcn_kernel(x_ref, adj_ref, w1_ref, b1_ref, w2_ref, b2_ref, w3_ref, b3_ref,
                out_ref, h_ref, u_ref, ss_ref):
    k = pl.program_id(0)
    t = pl.program_id(1)

    @pl.when(jnp.logical_and(k == 0, t == 0))
    def _init():
        xv = x_ref[...]
        ss0 = jnp.sum(xv * xv, axis=0, keepdims=True)
        inv = 1.0 / jnp.maximum(jnp.sqrt(ss0), 1e-12)
        h_ref[...] = xv * inv
        ss_ref[...] = jnp.zeros_like(ss_ref)

    @pl.when(jnp.logical_and(k > 0, t == 0))
    def _renorm():
        inv = 1.0 / jnp.maximum(jnp.sqrt(ss_ref[...]), 1e-12)
        h_ref[...] = u_ref[...] * inv
        ss_ref[...] = jnp.zeros_like(ss_ref)

    a = adj_ref[...]
    h = h_ref[...]
    row0 = pl.multiple_of(t * TILE, TILE)
    y = h_ref[pl.ds(row0, TILE), :] + jnp.dot(
        a, h, preferred_element_type=jnp.float32)
    u_ref[pl.ds(row0, TILE), :] = y
    ss_ref[...] += jnp.sum(y * y, axis=0, keepdims=True)

    @pl.when(jnp.logical_and(k == ITRS - 1, t == T - 1))
    def _mlp():
        inv = 1.0 / jnp.maximum(jnp.sqrt(ss_ref[...]), 1e-12)
        hf = u_ref[...] * inv
        t1 = jnp.maximum(
            jnp.dot(hf, w1_ref[...], preferred_element_type=jnp.float32)
            + b1_ref[...], 0.0)
        t2 = jnp.maximum(
            jnp.dot(t1, w2_ref[...], preferred_element_type=jnp.float32)
            + b2_ref[...], 0.0)
        out_ref[...] = jnp.dot(
            t2, w3_ref[...], preferred_element_type=jnp.float32) + b3_ref[...]


@jax.jit
def kernel(x, adj, W1, b1, W2, b2, W3, b3):
    x2d = x[0]
    out = pl.pallas_call(
        _gcn_kernel,
        grid=(ITRS, T),
        in_specs=[
            pl.BlockSpec((N, D_IN), lambda k, t: (0, 0)),
            pl.BlockSpec((TILE, N), lambda k, t: (t, 0)),
            pl.BlockSpec((D_IN, HIDDEN), lambda k, t: (0, 0)),
            pl.BlockSpec((1, HIDDEN), lambda k, t: (0, 0)),
            pl.BlockSpec((HIDDEN, HIDDEN), lambda k, t: (0, 0)),
            pl.BlockSpec((1, HIDDEN), lambda k, t: (0, 0)),
            pl.BlockSpec((HIDDEN, D_OUT), lambda k, t: (0, 0)),
            pl.BlockSpec((1, D_OUT), lambda k, t: (0, 0)),
        ],
        out_specs=pl.BlockSpec((N, D_OUT), lambda k, t: (0, 0)),
        out_shape=jax.ShapeDtypeStruct((N, D_OUT), jnp.float32),
        scratch_shapes=[
            pltpu.VMEM((N, D_IN), jnp.float32),
            pltpu.VMEM((N, D_IN), jnp.float32),
            pltpu.VMEM((1, D_IN), jnp.float32),
        ],
        compiler_params=pltpu.CompilerParams(
            dimension_semantics=("arbitrary", "arbitrary"),
        ),
    )(x2d, adj, W1.T, b1[None, :], W2.T, b2[None, :], W3.T, b3[None, :])
    return out[None, :, :]


# speedup vs baseline: 1.6683x; 1.2978x over previous
import jax
import jax.numpy as jnp
from jax.experimental import pallas as pl
from jax.experimental.pallas import tpu as pltpu

N = 4096
D_IN = 256
D_OUT = 256
HIDDEN = 128
ITRS = 5
TILE = 256
T = N // TILE


def _gcn_kernel(x_hbm, adj_hbm, w1_ref, b1_ref, w2_ref, b2_ref, w3_ref, b3_ref,
                out_ref, stage, adj_bf, u_ref, h_ref, ss_ref, dma_sems, x_sem):
    k = pl.program_id(0)
    t = pl.program_id(1)

    @pl.when(jnp.logical_and(k == 0, t == 0))
    def _start():
        pltpu.make_async_copy(x_hbm, u_ref, x_sem).start()
        pltpu.make_async_copy(
            adj_hbm.at[pl.ds(0, TILE), :], stage.at[0], dma_sems.at[0]).start()
        pltpu.make_async_copy(
            adj_hbm.at[pl.ds(TILE, TILE), :], stage.at[1], dma_sems.at[1]).start()
        pltpu.make_async_copy(x_hbm, u_ref, x_sem).wait()
        xv = u_ref[...]
        ss0 = jnp.sum(xv * xv, axis=0, keepdims=True)
        inv = 1.0 / jnp.maximum(jnp.sqrt(ss0), 1e-12)
        h_ref[...] = (xv * inv).astype(jnp.bfloat16)
        ss_ref[...] = jnp.zeros_like(ss_ref)

    @pl.when(jnp.logical_and(k > 0, t == 0))
    def _renorm():
        inv = 1.0 / jnp.maximum(jnp.sqrt(ss_ref[...]), 1e-12)
        h_ref[...] = (u_ref[...] * inv).astype(jnp.bfloat16)
        ss_ref[...] = jnp.zeros_like(ss_ref)

    row0 = pl.multiple_of(t * TILE, TILE)

    @pl.when(k == 0)
    def _fill():
        slot = jax.lax.rem(t, 2)
        pltpu.make_async_copy(
            adj_hbm.at[pl.ds(row0, TILE), :], stage.at[slot],
            dma_sems.at[slot]).wait()
        adj_bf[pl.ds(row0, TILE), :] = stage[slot].astype(jnp.bfloat16)

        @pl.when(t + 2 < T)
        def _next():
            nxt = pl.multiple_of((t + 2) * TILE, TILE)
            pltpu.make_async_copy(
                adj_hbm.at[pl.ds(nxt, TILE), :], stage.at[slot],
                dma_sems.at[slot]).start()

    a = adj_bf[pl.ds(row0, TILE), :]
    h = h_ref[...]
    y = jnp.dot(a, h, preferred_element_type=jnp.float32) \
        + h_ref[pl.ds(row0, TILE), :].astype(jnp.float32)
    u_ref[pl.ds(row0, TILE), :] = y
    ss_ref[...] += jnp.sum(y * y, axis=0, keepdims=True)

    @pl.when(jnp.logical_and(k == ITRS - 1, t == T - 1))
    def _mlp():
        inv = 1.0 / jnp.maximum(jnp.sqrt(ss_ref[...]), 1e-12)
        hf = u_ref[...] * inv
        t1 = jnp.maximum(
            jnp.dot(hf, w1_ref[...], preferred_element_type=jnp.float32)
            + b1_ref[...], 0.0)
        t2 = jnp.maximum(
            jnp.dot(t1, w2_ref[...], preferred_element_type=jnp.float32)
            + b2_ref[...], 0.0)
        out_ref[...] = jnp.dot(
            t2, w3_ref[...], preferred_element_type=jnp.float32) + b3_ref[...]


@jax.jit
def kernel(x, adj, W1, b1, W2, b2, W3, b3):
    x2d = x[0]
    out = pl.pallas_call(
        _gcn_kernel,
        grid=(ITRS, T),
        in_specs=[
            pl.BlockSpec(memory_space=pl.ANY),
            pl.BlockSpec(memory_space=pl.ANY),
            pl.BlockSpec((D_IN, HIDDEN), lambda k, t: (0, 0)),
            pl.BlockSpec((1, HIDDEN), lambda k, t: (0, 0)),
            pl.BlockSpec((HIDDEN, HIDDEN), lambda k, t: (0, 0)),
            pl.BlockSpec((1, HIDDEN), lambda k, t: (0, 0)),
            pl.BlockSpec((HIDDEN, D_OUT), lambda k, t: (0, 0)),
            pl.BlockSpec((1, D_OUT), lambda k, t: (0, 0)),
        ],
        out_specs=pl.BlockSpec((N, D_OUT), lambda k, t: (0, 0)),
        out_shape=jax.ShapeDtypeStruct((N, D_OUT), jnp.float32),
        scratch_shapes=[
            pltpu.VMEM((2, TILE, N), jnp.float32),
            pltpu.VMEM((N, N), jnp.bfloat16),
            pltpu.VMEM((N, D_IN), jnp.float32),
            pltpu.VMEM((N, D_IN), jnp.bfloat16),
            pltpu.VMEM((1, D_IN), jnp.float32),
            pltpu.SemaphoreType.DMA((2,)),
            pltpu.SemaphoreType.DMA,
        ],
        compiler_params=pltpu.CompilerParams(
            dimension_semantics=("arbitrary", "arbitrary"),
            vmem_limit_bytes=64 * 1024 * 1024,
        ),
    )(x2d, adj, W1.T, b1[None, :], W2.T, b2[None, :], W3.T, b3[None, :])
    return out[None, :, :]
